# Initial kernel scaffold; baseline (speedup 1.0000x reference)
import jax
import jax.numpy as jnp
from jax.experimental import pallas as pl


def kernel(pos, x, batch, W_head, b_head, W_blocks, b_blocks, W1, b1, W2, b2, W3, b3):
    raise NotImplementedError("write your pallas kernel here")


# speedup vs baseline: 1.1151x; 1.1151x over previous
import functools
import jax
import jax.numpy as jnp
from jax import lax
from jax.experimental import pallas as pl
from jax.experimental.pallas import tpu as pltpu

N = 4096
K = 16
RBLK = 256
NBLK = N // RBLK


def _d2_body(f_ref, sq_ref, b_ref, o_ref):
    i = pl.program_id(0)
    f = f_ref[...]
    fr = f_ref[pl.ds(i * RBLK, RBLK), :]
    g = lax.dot_general(fr, f, (((1,), (1,)), ((), ())))
    sqf = sq_ref[0, :]
    sqr = sq_ref[0, pl.ds(i * RBLK, RBLK)]
    d2 = sqr[:, None] + sqf[None, :] - 2.0 * g
    b = b_ref[0, :]
    br = b_ref[0, pl.ds(i * RBLK, RBLK)]
    col = lax.broadcasted_iota(jnp.int32, (RBLK, N), 1)
    row = lax.broadcasted_iota(jnp.int32, (RBLK, N), 0) + i * RBLK
    bad = (br[:, None] != b[None, :]) | (col == row)
    o_ref[...] = jnp.where(bad, jnp.inf, d2)


def _d2(f, batch):
    sq = jnp.sum(f * f, axis=1)
    return pl.pallas_call(
        _d2_body,
        grid=(NBLK,),
        in_specs=[
            pl.BlockSpec(f.shape, lambda i: (0, 0)),
            pl.BlockSpec((1, N), lambda i: (0, 0)),
            pl.BlockSpec((1, N), lambda i: (0, 0)),
        ],
        out_specs=pl.BlockSpec((RBLK, N), lambda i: (i, 0)),
        out_shape=jax.ShapeDtypeStruct((N, N), jnp.float32),
    )(f, sq.reshape(1, N), batch.reshape(1, N))


def _edge_body(e_ref, w_ref, b_ref, o_ref):
    c = w_ref.shape[1]
    h = jax.nn.relu(jnp.dot(e_ref[...], w_ref[...]) + b_ref[0, :][None, :])
    o_ref[...] = jnp.max(h.reshape(RBLK, K, c), axis=1)


def _edge_conv(x, nbr, W, b):
    xj = jnp.take(x, nbr, axis=0)
    xi = jnp.broadcast_to(x[:, None, :], xj.shape)
    e = jnp.concatenate([xi, xj - xi], axis=-1).reshape(N * K, -1)
    c = W.shape[1]
    return pl.pallas_call(
        _edge_body,
        grid=(NBLK,),
        in_specs=[
            pl.BlockSpec((RBLK * K, e.shape[1]), lambda i: (i, 0)),
            pl.BlockSpec(W.shape, lambda i: (0, 0)),
            pl.BlockSpec((1, c), lambda i: (0, 0)),
        ],
        out_specs=pl.BlockSpec((RBLK, c), lambda i: (i, 0)),
        out_shape=jax.ShapeDtypeStruct((N, c), jnp.float32),
    )(e, W, b.reshape(1, c))


def _mlp_body(o_ref, w1_ref, b1_ref, w2_ref, b2_ref, w3_ref, b3_ref, out_ref):
    h = jax.nn.relu(jnp.dot(o_ref[...], w1_ref[...]) + b1_ref[0, :][None, :])
    h = jax.nn.relu(jnp.dot(h, w2_ref[...]) + b2_ref[0, :][None, :])
    out_ref[...] = jnp.dot(h, w3_ref[...]) + b3_ref[0, :][None, :]


def _mlp(o, W1, b1, W2, b2, W3, b3):
    return pl.pallas_call(
        _mlp_body,
        out_shape=jax.ShapeDtypeStruct((N, W3.shape[1]), jnp.float32),
    )(o, W1, b1.reshape(1, -1), W2, b2.reshape(1, -1), W3, b3.reshape(1, -1))


def _select(d2, k, d):
    _, idx = lax.top_k(-d2, k * d)
    return idx[:, ::d]


def kernel(pos, x, batch, W_head, b_head, W_blocks, b_blocks, W1, b1, W2, b2, W3, b3):
    feats0 = jnp.concatenate([pos, x], axis=1)
    nbr = _select(_d2(pos, batch), K, 1)
    f = _edge_conv(feats0, nbr, W_head, b_head)
    feats = [f]
    for i in range(7):
        dil = 1 + i
        nbr = _select(_d2(feats[-1], batch), K, dil)
        feats.append(_edge_conv(feats[-1], nbr, W_blocks[i], b_blocks[i]))
    out = jnp.stack(feats, axis=0)
    out = jnp.transpose(out, (1, 0, 2)).reshape(N, -1)
    return _mlp(out, W1, b1, W2, b2, W3, b3)
